# Initial kernel scaffold; baseline (speedup 1.0000x reference)
import jax
import jax.numpy as jnp
from jax.experimental import pallas as pl


def kernel(input_, weight):
    raise NotImplementedError("write your pallas kernel here")


# speedup vs baseline: 3.2977x; 3.2977x over previous
import functools

import jax
import jax.numpy as jnp
from jax import lax
from jax.experimental import pallas as pl
from jax.experimental.pallas import tpu as pltpu
from jax.experimental.pallas import tpu_sc as plsc

NUM_EMBEDDINGS = 100000
D = 128
B = 4096 * 50
NC = 2
NS = 16
NW = NC * NS
IPR = 128
B_PER_W = B // NW
F = 5
OUTER = B_PER_W // (F * IPR)


def _emb_body(idx_hbm, table_hbm, out_hbm, idx_v, rows_v, sem):
    wid = lax.axis_index("s") * NC + lax.axis_index("c")
    base = wid * B_PER_W
    pltpu.sync_copy(idx_hbm.at[pl.ds(base, B_PER_W)], idx_v)

    def step(o, carry):
        copies = []
        for j in range(F):
            copies.append(pltpu.async_copy(
                table_hbm.at[idx_v.at[pl.ds((o * F + j) * IPR, IPR)]],
                rows_v.at[pl.ds(j * IPR, IPR)],
                sem,
            ))
        for c in copies:
            c.wait()
        pltpu.sync_copy(rows_v, out_hbm.at[pl.ds(base + o * (F * IPR), F * IPR)])
        return carry

    lax.fori_loop(0, OUTER, step, 0)


@jax.jit
def _embedding_lookup(idx2d, weight):
    mesh = plsc.VectorSubcoreMesh(core_axis_name="c", subcore_axis_name="s")
    fn = functools.partial(
        pl.kernel,
        mesh=mesh,
        out_type=jax.ShapeDtypeStruct((B, D), jnp.float32),
        scratch_types=[
            pltpu.VMEM((B_PER_W,), jnp.int32),
            pltpu.VMEM((F * IPR, D), jnp.float32),
            pltpu.SemaphoreType.DMA,
        ],
    )(_emb_body)
    return fn(idx2d, weight)


def kernel(input_, weight):
    idx = input_.astype(jnp.int32).reshape(B)
    out = _embedding_lookup(idx, weight)
    return out.reshape(input_.shape[0], input_.shape[1], D)


# speedup vs baseline: 3.3372x; 1.0120x over previous
import functools

import jax
import jax.numpy as jnp
from jax import lax
from jax.experimental import pallas as pl
from jax.experimental.pallas import tpu as pltpu
from jax.experimental.pallas import tpu_sc as plsc

NUM_EMBEDDINGS = 100000
D = 128
B = 4096 * 50
NC = 2
NS = 16
NW = NC * NS
IPR = 128
B_PER_W = B // NW
NGATH = B_PER_W // IPR
G = 3
GROUPS = [G] * (NGATH // G) + ([NGATH % G] if NGATH % G else [])


def _emb_body(idx_hbm, table_hbm, out_hbm, idx_v, rows_a, rows_b, gsa, gsb, osa, osb):
    wid = lax.axis_index("s") * NC + lax.axis_index("c")
    base = wid * B_PER_W
    pltpu.sync_copy(idx_hbm.at[pl.ds(base, B_PER_W)], idx_v)

    bufs = (rows_a, rows_b)
    gsems = (gsa, gsb)
    osems = (osa, osb)
    starts = [sum(GROUPS[:g]) for g in range(len(GROUPS))]

    gath = {}
    outc = {}
    S = len(GROUPS)
    for g in range(S):
        b = g % 2
        if g >= 2:
            outc[g - 2].wait()
        gath[g] = [
            pltpu.async_copy(
                table_hbm.at[idx_v.at[pl.ds((starts[g] + j) * IPR, IPR)]],
                bufs[b].at[pl.ds(j * IPR, IPR)],
                gsems[b],
            )
            for j in range(GROUPS[g])
        ]
        if g >= 1:
            for c in gath[g - 1]:
                c.wait()
            n = GROUPS[g - 1] * IPR
            outc[g - 1] = pltpu.async_copy(
                bufs[1 - b].at[pl.ds(0, n)],
                out_hbm.at[pl.ds(base + starts[g - 1] * IPR, n)],
                osems[1 - b],
            )
    for c in gath[S - 1]:
        c.wait()
    outc[S - 2].wait()
    n = GROUPS[S - 1] * IPR
    outc[S - 1] = pltpu.async_copy(
        bufs[(S - 1) % 2].at[pl.ds(0, n)],
        out_hbm.at[pl.ds(base + starts[S - 1] * IPR, n)],
        osems[(S - 1) % 2],
    )
    outc[S - 1].wait()


@jax.jit
def _embedding_lookup(idx2d, weight):
    mesh = plsc.VectorSubcoreMesh(core_axis_name="c", subcore_axis_name="s")
    fn = functools.partial(
        pl.kernel,
        mesh=mesh,
        out_type=jax.ShapeDtypeStruct((B, D), jnp.float32),
        scratch_types=[
            pltpu.VMEM((B_PER_W,), jnp.int32),
            pltpu.VMEM((G * IPR, D), jnp.float32),
            pltpu.VMEM((G * IPR, D), jnp.float32),
            pltpu.SemaphoreType.DMA,
            pltpu.SemaphoreType.DMA,
            pltpu.SemaphoreType.DMA,
            pltpu.SemaphoreType.DMA,
        ],
    )(_emb_body)
    return fn(idx2d, weight)


def kernel(input_, weight):
    idx = input_.astype(jnp.int32).reshape(B)
    out = _embedding_lookup(idx, weight)
    return out.reshape(input_.shape[0], input_.shape[1], D)


# speedup vs baseline: 5.7647x; 1.7274x over previous
import functools

import jax
import jax.numpy as jnp
from jax import lax
from jax.experimental import pallas as pl
from jax.experimental.pallas import tpu as pltpu
from jax.experimental.pallas import tpu_sc as plsc

NUM_EMBEDDINGS = 100000
NB = 4096
SL = 50
D = 128
B = NB * SL
NC = 2
NS = 16
NW = NC * NS
B_PER_W = B // NW
NB_PER_W = NB // NW
BB = 8
ROWS_PER_GROUP = BB * SL
S = NB_PER_W // BB
CHUNKS = [(0, 128), (128, 128), (256, 128), (384, 16)]


def _emb_body(idx_hbm, table_hbm, out_hbm, idx_v, rows_a, rows_b, gsa, gsb, osa, osb):
    wid = lax.axis_index("s") * NC + lax.axis_index("c")
    base = wid * B_PER_W
    b0 = wid * NB_PER_W
    pltpu.sync_copy(idx_hbm.at[pl.ds(base, B_PER_W)], idx_v)

    bufs = (rows_a, rows_b)
    gsems = (gsa, gsb)
    osems = (osa, osb)

    gath = {}
    outc = {}
    for g in range(S):
        bsel = g % 2
        if g >= 2:
            for c in outc[g - 2]:
                c.wait()
        gath[g] = [
            pltpu.async_copy(
                table_hbm.at[idx_v.at[pl.ds(g * ROWS_PER_GROUP + off, n)]],
                bufs[bsel].at[pl.ds(off, n)],
                gsems[bsel],
            )
            for off, n in CHUNKS
        ]
        if g >= 1:
            for c in gath[g - 1]:
                c.wait()
            outc[g - 1] = [
                pltpu.async_copy(
                    bufs[1 - bsel].at[pl.ds(j * SL, SL)],
                    out_hbm.at[b0 + (g - 1) * BB + j],
                    osems[1 - bsel],
                )
                for j in range(BB)
            ]
    for c in gath[S - 1]:
        c.wait()
    for c in outc[S - 2]:
        c.wait()
    last = [
        pltpu.async_copy(
            bufs[(S - 1) % 2].at[pl.ds(j * SL, SL)],
            out_hbm.at[b0 + (S - 1) * BB + j],
            osems[(S - 1) % 2],
        )
        for j in range(BB)
    ]
    for c in last:
        c.wait()


@jax.jit
def _embedding_lookup(idx, weight):
    mesh = plsc.VectorSubcoreMesh(core_axis_name="c", subcore_axis_name="s")
    fn = functools.partial(
        pl.kernel,
        mesh=mesh,
        out_type=jax.ShapeDtypeStruct((NB, SL, D), jnp.float32),
        scratch_types=[
            pltpu.VMEM((B_PER_W,), jnp.int32),
            pltpu.VMEM((ROWS_PER_GROUP, D), jnp.float32),
            pltpu.VMEM((ROWS_PER_GROUP, D), jnp.float32),
            pltpu.SemaphoreType.DMA,
            pltpu.SemaphoreType.DMA,
            pltpu.SemaphoreType.DMA,
            pltpu.SemaphoreType.DMA,
        ],
    )(_emb_body)
    return fn(idx, weight)


def kernel(input_, weight):
    idx = input_.astype(jnp.int32).reshape(B)
    return _embedding_lookup(idx, weight)


# speedup vs baseline: 10.2486x; 1.7778x over previous
import functools

import jax
import jax.numpy as jnp
from jax import lax
from jax.experimental import pallas as pl
from jax.experimental.pallas import tpu as pltpu
from jax.experimental.pallas import tpu_sc as plsc

NUM_EMBEDDINGS = 100000
NB = 4096
SL = 50
D = 128
B = NB * SL
NC = 2
NS = 16
NW = NC * NS
B_PER_W = B // NW
IPR = 128
NGATH = B_PER_W // IPR
G = 3
GROUPS = [G] * (NGATH // G) + ([NGATH % G] if NGATH % G else [])


def _emb_body(idx_hbm, table_hbm, out_hbm, idx_v, rows_a, rows_b, gsa, gsb, osa, osb):
    wid = lax.axis_index("s") * NC + lax.axis_index("c")
    base = wid * B_PER_W
    pltpu.sync_copy(idx_hbm.at[pl.ds(base, B_PER_W)], idx_v)

    bufs = (rows_a, rows_b)
    gsems = (gsa, gsb)
    osems = (osa, osb)
    starts = [sum(GROUPS[:g]) for g in range(len(GROUPS))]

    gath = {}
    outc = {}
    S = len(GROUPS)
    for g in range(S):
        b = g % 2
        if g >= 2:
            outc[g - 2].wait()
        gath[g] = [
            pltpu.async_copy(
                table_hbm.at[idx_v.at[pl.ds((starts[g] + j) * IPR, IPR)]],
                bufs[b].at[pl.ds(j * IPR, IPR)],
                gsems[b],
            )
            for j in range(GROUPS[g])
        ]
        if g >= 1:
            for c in gath[g - 1]:
                c.wait()
            n = GROUPS[g - 1] * IPR
            outc[g - 1] = pltpu.async_copy(
                bufs[1 - b].at[pl.ds(0, n)],
                out_hbm.at[pl.ds(base + starts[g - 1] * IPR, n)],
                osems[1 - b],
            )
    for c in gath[S - 1]:
        c.wait()
    outc[S - 2].wait()
    n = GROUPS[S - 1] * IPR
    outc[S - 1] = pltpu.async_copy(
        bufs[(S - 1) % 2].at[pl.ds(0, n)],
        out_hbm.at[pl.ds(base + starts[S - 1] * IPR, n)],
        osems[(S - 1) % 2],
    )
    outc[S - 1].wait()


@jax.jit
def _embedding_lookup(idx, weight):
    mesh = plsc.VectorSubcoreMesh(core_axis_name="c", subcore_axis_name="s")
    fn = functools.partial(
        pl.kernel,
        mesh=mesh,
        out_type=jax.ShapeDtypeStruct((B, D), jnp.float32),
        scratch_types=[
            pltpu.VMEM((B_PER_W,), jnp.int32),
            pltpu.VMEM((G * IPR, D), jnp.float32),
            pltpu.VMEM((G * IPR, D), jnp.float32),
            pltpu.SemaphoreType.DMA,
            pltpu.SemaphoreType.DMA,
            pltpu.SemaphoreType.DMA,
            pltpu.SemaphoreType.DMA,
        ],
    )(_emb_body)
    return fn(idx, weight)


def kernel(input_, weight):
    idx = input_.astype(jnp.int32).T.reshape(B)
    out = _embedding_lookup(idx, weight)
    return out.reshape(SL, NB, D).transpose(1, 0, 2)
